# baseline (speedup 1.0000x reference)
import jax
import jax.numpy as jnp
from jax.experimental import pallas as pl
from jax.experimental.pallas import tpu as pltpu

P = 49
R = 256
C = 256
CM = 128
PR = P * R
TIMES = 2
NEA = 3
FAR = 3


def _relation_kernel(p_ref, pt_ref, x0_ref, w1_ref, w2_ref, b1_ref, b2_ref,
                     out_ref, xcat_ref, a1f_ref, scat_ref, s_ref, sem):
    n = pl.program_id(0)
    cp = pltpu.make_async_copy(x0_ref.at[n], out_ref.at[0], sem)
    cp.start()
    p = p_ref[0]
    pt = pt_ref[0]
    x1c, y1c, x2c, y2c = p[:, 0:1], p[:, 1:2], p[:, 2:3], p[:, 3:4]
    x1r, y1r, x2r, y2r = pt[0:1, :], pt[1:2, :], pt[2:3, :], pt[3:4, :]
    wc = x2c - x1c + 1.0
    hc = y2c - y1c + 1.0
    wr = x2r - x1r + 1.0
    hr = y2r - y1r + 1.0
    cxc = x1c + 0.5 * wc
    cyc = y1c + 0.5 * hc
    cxr = x1r + 0.5 * wr
    cyr = y1r + 0.5 * hr
    sq = (cxc - cxr) ** 2 + (cyc - cyr) ** 2
    pos = sq > 0
    dist = jnp.where(pos, jnp.sqrt(jnp.where(pos, sq, 1.0)), 0.0)
    iw = jnp.maximum(jnp.minimum(x2r, x2c) - jnp.maximum(x1r, x1c) + 1.0, 0.0)
    ih = jnp.maximum(jnp.minimum(y2r, y2c) - jnp.maximum(y1r, y1c) + 1.0, 0.0)
    inter = iw * ih
    union = hc * wc + hr * wr - inter
    ov = inter / union
    dmax = jnp.max(dist)
    dist_nea = jnp.where(ov != 0.0, dist, 0.0)
    dist_far = jnp.where(ov == 0.0, dist, dmax)

    cols = jax.lax.broadcasted_iota(jnp.int32, (R, R), 1)
    ids = []
    ws = []
    a = dist_nea
    for _ in range(NEA):
        m = jnp.max(a, axis=1, keepdims=True)
        idx = jnp.min(jnp.where(a == m, cols, R), axis=1, keepdims=True)
        ids.append(idx)
        ws.append(m)
        a = jnp.where(cols == idx, -jnp.inf, a)
    b = dist_far
    for _ in range(FAR):
        m = jnp.min(b, axis=1, keepdims=True)
        idx = jnp.min(jnp.where(b == m, cols, R), axis=1, keepdims=True)
        ids.append(idx)
        ws.append(m)
        b = jnp.where(cols == idx, jnp.inf, b)

    w6 = jnp.concatenate(ws, axis=1) * 0.01
    mx = jnp.max(w6, axis=1, keepdims=True)
    e = jnp.exp(w6 - mx)
    w6 = e / jnp.sum(e, axis=1, keepdims=True)

    s = jnp.zeros((R, R), jnp.float32)
    for k in range(6):
        s = s + jnp.where(cols == ids[k], w6[:, k:k + 1], 0.0)
    s_ref[...] = s.astype(jnp.bfloat16)

    cp.wait()

    zc = jnp.zeros((R, C), jnp.bfloat16)
    zm = jnp.zeros((R, CM), jnp.bfloat16)

    for _t in range(TIMES):
        xbf = out_ref[0]
        xcat_ref[:, C:2 * C] = xbf
        xcat_ref[pl.ds(R, PR - R), 0:C] = xbf[0:PR - R]
        xcat_ref[0:R, 0:C] = zc
        for yy in range(6):
            xcat_ref[pl.ds((yy + 1) * 7 * R, R), 0:C] = zc
        xcat_ref[pl.ds(0, PR - R), 2 * C:3 * C] = xbf[R:PR]
        xcat_ref[pl.ds(PR - R, R), 2 * C:3 * C] = zc
        for yy in range(1, 7):
            xcat_ref[pl.ds(yy * 7 * R - R, R), 2 * C:3 * C] = zc

        a1f_ref[...] = b1_ref[...] + jnp.dot(
            xcat_ref[...], w1_ref[1], preferred_element_type=jnp.float32)
        a1f_ref[pl.ds(7 * R, PR - 7 * R), :] = (
            a1f_ref[pl.ds(7 * R, PR - 7 * R), :]
            + jnp.dot(xcat_ref[pl.ds(0, PR - 7 * R), :], w1_ref[0],
                      preferred_element_type=jnp.float32))
        a1f_ref[pl.ds(0, PR - 7 * R), :] = (
            a1f_ref[pl.ds(0, PR - 7 * R), :]
            + jnp.dot(xcat_ref[pl.ds(7 * R, PR - 7 * R), :], w1_ref[2],
                      preferred_element_type=jnp.float32))

        scat_ref[0:R, 0:CM] = zm
        scat_ref[pl.ds(PR - R, R), 2 * CM:3 * CM] = zm
        for pp in range(P):
            x = pp % 7
            v = jnp.dot(s_ref[...],
                        a1f_ref[pl.ds(pp * R, R), :].astype(jnp.bfloat16),
                        preferred_element_type=jnp.float32).astype(jnp.bfloat16)
            scat_ref[pl.ds(pp * R, R), CM:2 * CM] = v
            if pp + 1 < P:
                scat_ref[pl.ds((pp + 1) * R, R), 0:CM] = (
                    zm if x == 6 else v)
            if pp - 1 >= 0:
                scat_ref[pl.ds((pp - 1) * R, R), 2 * CM:3 * CM] = (
                    zm if x == 0 else v)

        def rmw2(o, nn, wi, so, bias):
            h = nn // 2
            for oo, hh in ((o, h), (o + h, nn - h)):
                d = jnp.dot(scat_ref[pl.ds(oo - o + so, hh), :], w2_ref[wi],
                            preferred_element_type=jnp.float32)
                if bias:
                    d = d + b2_ref[...]
                out_ref[0, pl.ds(oo, hh), :] = (
                    out_ref[0, pl.ds(oo, hh), :] + d).astype(jnp.bfloat16)

        rmw2(0, PR, 1, 0, True)
        rmw2(7 * R, PR - 7 * R, 0, 0, False)
        rmw2(0, PR - 7 * R, 2, 7 * R, False)


def kernel(proposals, pooled_feat, W1, b1, W2, b2):
    n_img = proposals.shape[0]
    pf = pooled_feat.reshape(n_img, R, C, 7, 7)
    x0 = (pf.transpose(0, 3, 4, 1, 2).reshape(n_img, PR, C)
          .astype(jnp.bfloat16))
    w1t = jnp.pad(W1.transpose(2, 3, 1, 0).reshape(3, 3, C, 81),
                  ((0, 0), (0, 0), (0, 0), (0, CM - 81)))
    w1c = w1t.transpose(0, 1, 2, 3).reshape(3, 3 * C, CM).astype(jnp.bfloat16)
    w2t = jnp.pad(W2.transpose(2, 3, 1, 0).reshape(3, 3, 81, C),
                  ((0, 0), (0, 0), (0, CM - 81), (0, 0)))
    w2c = w2t.reshape(3, 3 * CM, C).astype(jnp.bfloat16)
    b1p = jnp.pad(b1, (0, CM - 81)).reshape(1, CM)
    b2p = b2.reshape(1, C)
    pt = proposals.transpose(0, 2, 1)

    out = pl.pallas_call(
        _relation_kernel,
        grid=(n_img,),
        in_specs=[
            pl.BlockSpec((1, R, 4), lambda n: (n, 0, 0)),
            pl.BlockSpec((1, 4, R), lambda n: (n, 0, 0)),
            pl.BlockSpec(memory_space=pl.ANY),
            pl.BlockSpec((3, 3 * C, CM), lambda n: (0, 0, 0)),
            pl.BlockSpec((3, 3 * CM, C), lambda n: (0, 0, 0)),
            pl.BlockSpec((1, CM), lambda n: (0, 0)),
            pl.BlockSpec((1, C), lambda n: (0, 0)),
        ],
        out_specs=pl.BlockSpec((1, PR, C), lambda n: (n, 0, 0)),
        out_shape=jax.ShapeDtypeStruct((n_img, PR, C), jnp.bfloat16),
        scratch_shapes=[
            pltpu.VMEM((PR, 3 * C), jnp.bfloat16),
            pltpu.VMEM((PR, CM), jnp.float32),
            pltpu.VMEM((PR, 3 * CM), jnp.bfloat16),
            pltpu.VMEM((R, R), jnp.bfloat16),
            pltpu.SemaphoreType.DMA,
        ],
    )(proposals, pt, x0, w1c, w2c, b1p, b2p)

    y = out.reshape(n_img, 7, 7, R, C).transpose(0, 3, 4, 1, 2)
    return y.reshape(n_img * R, C, 7, 7).astype(jnp.float32)


# speedup vs baseline: 1.0448x; 1.0448x over previous
import jax
import jax.numpy as jnp
from jax.experimental import pallas as pl
from jax.experimental.pallas import tpu as pltpu

P = 49
R = 256
C = 256
CM = 128
PR = P * R
TIMES = 2
NEA = 3
FAR = 3


def _relation_kernel(p_ref, pt_ref, x0_ref, w1_ref, w2_ref, b1_ref, b2_ref,
                     out_ref, xcat_ref, a1f_ref, scat_ref, s_ref, sem):
    n = pl.program_id(0)
    cp = pltpu.make_async_copy(x0_ref.at[n], out_ref.at[0], sem)
    cp.start()
    p = p_ref[0]
    pt = pt_ref[0]
    x1c, y1c, x2c, y2c = p[:, 0:1], p[:, 1:2], p[:, 2:3], p[:, 3:4]
    x1r, y1r, x2r, y2r = pt[0:1, :], pt[1:2, :], pt[2:3, :], pt[3:4, :]
    wc = x2c - x1c + 1.0
    hc = y2c - y1c + 1.0
    wr = x2r - x1r + 1.0
    hr = y2r - y1r + 1.0
    cxc = x1c + 0.5 * wc
    cyc = y1c + 0.5 * hc
    cxr = x1r + 0.5 * wr
    cyr = y1r + 0.5 * hr
    sq = (cxc - cxr) ** 2 + (cyc - cyr) ** 2
    pos = sq > 0
    dist = jnp.where(pos, jnp.sqrt(jnp.where(pos, sq, 1.0)), 0.0)
    iw = jnp.maximum(jnp.minimum(x2r, x2c) - jnp.maximum(x1r, x1c) + 1.0, 0.0)
    ih = jnp.maximum(jnp.minimum(y2r, y2c) - jnp.maximum(y1r, y1c) + 1.0, 0.0)
    inter = iw * ih
    union = hc * wc + hr * wr - inter
    ov = inter / union
    dmax = jnp.max(dist)
    dist_nea = jnp.where(ov != 0.0, dist, 0.0)
    dist_far = jnp.where(ov == 0.0, dist, dmax)

    cols = jax.lax.broadcasted_iota(jnp.int32, (R, R), 1)
    ids = []
    ws = []
    a = dist_nea
    for _ in range(NEA):
        m = jnp.max(a, axis=1, keepdims=True)
        idx = jnp.min(jnp.where(a == m, cols, R), axis=1, keepdims=True)
        ids.append(idx)
        ws.append(m)
        a = jnp.where(cols == idx, -jnp.inf, a)
    b = dist_far
    for _ in range(FAR):
        m = jnp.min(b, axis=1, keepdims=True)
        idx = jnp.min(jnp.where(b == m, cols, R), axis=1, keepdims=True)
        ids.append(idx)
        ws.append(m)
        b = jnp.where(cols == idx, jnp.inf, b)

    w6 = jnp.concatenate(ws, axis=1) * 0.01
    mx = jnp.max(w6, axis=1, keepdims=True)
    e = jnp.exp(w6 - mx)
    w6 = e / jnp.sum(e, axis=1, keepdims=True)

    s = jnp.zeros((R, R), jnp.float32)
    for k in range(6):
        s = s + jnp.where(cols == ids[k], w6[:, k:k + 1], 0.0)
    s_ref[...] = s.astype(jnp.bfloat16)

    cp.wait()

    zc = jnp.zeros((R, C), jnp.bfloat16)
    zm = jnp.zeros((R, CM), jnp.bfloat16)

    for _t in range(TIMES):
        xbf = out_ref[0].astype(jnp.bfloat16)
        xcat_ref[:, C:2 * C] = xbf
        xcat_ref[pl.ds(R, PR - R), 0:C] = xbf[0:PR - R]
        xcat_ref[0:R, 0:C] = zc
        for yy in range(6):
            xcat_ref[pl.ds((yy + 1) * 7 * R, R), 0:C] = zc
        xcat_ref[pl.ds(0, PR - R), 2 * C:3 * C] = xbf[R:PR]
        xcat_ref[pl.ds(PR - R, R), 2 * C:3 * C] = zc
        for yy in range(1, 7):
            xcat_ref[pl.ds(yy * 7 * R - R, R), 2 * C:3 * C] = zc

        a1f_ref[...] = b1_ref[...] + jnp.dot(
            xcat_ref[...], w1_ref[1], preferred_element_type=jnp.float32)
        a1f_ref[pl.ds(7 * R, PR - 7 * R), :] = (
            a1f_ref[pl.ds(7 * R, PR - 7 * R), :]
            + jnp.dot(xcat_ref[pl.ds(0, PR - 7 * R), :], w1_ref[0],
                      preferred_element_type=jnp.float32))
        a1f_ref[pl.ds(0, PR - 7 * R), :] = (
            a1f_ref[pl.ds(0, PR - 7 * R), :]
            + jnp.dot(xcat_ref[pl.ds(7 * R, PR - 7 * R), :], w1_ref[2],
                      preferred_element_type=jnp.float32))

        scat_ref[0:R, 0:CM] = zm
        scat_ref[pl.ds(PR - R, R), 2 * CM:3 * CM] = zm
        for pp in range(P):
            x = pp % 7
            v = jnp.dot(s_ref[...],
                        a1f_ref[pl.ds(pp * R, R), :].astype(jnp.bfloat16),
                        preferred_element_type=jnp.float32).astype(jnp.bfloat16)
            scat_ref[pl.ds(pp * R, R), CM:2 * CM] = v
            if pp + 1 < P:
                scat_ref[pl.ds((pp + 1) * R, R), 0:CM] = (
                    zm if x == 6 else v)
            if pp - 1 >= 0:
                scat_ref[pl.ds((pp - 1) * R, R), 2 * CM:3 * CM] = (
                    zm if x == 0 else v)

        out_ref[0] = out_ref[0] + b2_ref[...] + jnp.dot(
            scat_ref[...], w2_ref[1], preferred_element_type=jnp.float32)
        out_ref[0, pl.ds(7 * R, PR - 7 * R), :] = (
            out_ref[0, pl.ds(7 * R, PR - 7 * R), :]
            + jnp.dot(scat_ref[pl.ds(0, PR - 7 * R), :], w2_ref[0],
                      preferred_element_type=jnp.float32))
        out_ref[0, pl.ds(0, PR - 7 * R), :] = (
            out_ref[0, pl.ds(0, PR - 7 * R), :]
            + jnp.dot(scat_ref[pl.ds(7 * R, PR - 7 * R), :], w2_ref[2],
                      preferred_element_type=jnp.float32))


def kernel(proposals, pooled_feat, W1, b1, W2, b2):
    n_img = proposals.shape[0]
    pf = pooled_feat.reshape(n_img, R, C, 7, 7)
    x0 = pf.transpose(0, 3, 4, 1, 2).reshape(n_img, PR, C)
    w1t = jnp.pad(W1.transpose(2, 3, 1, 0).reshape(3, 3, C, 81),
                  ((0, 0), (0, 0), (0, 0), (0, CM - 81)))
    w1c = w1t.reshape(3, 3 * C, CM).astype(jnp.bfloat16)
    w2t = jnp.pad(W2.transpose(2, 3, 1, 0).reshape(3, 3, 81, C),
                  ((0, 0), (0, 0), (0, CM - 81), (0, 0)))
    w2c = w2t.reshape(3, 3 * CM, C).astype(jnp.bfloat16)
    b1p = jnp.pad(b1, (0, CM - 81)).reshape(1, CM)
    b2p = b2.reshape(1, C)
    pt = proposals.transpose(0, 2, 1)

    out = pl.pallas_call(
        _relation_kernel,
        grid=(n_img,),
        in_specs=[
            pl.BlockSpec((1, R, 4), lambda n: (n, 0, 0)),
            pl.BlockSpec((1, 4, R), lambda n: (n, 0, 0)),
            pl.BlockSpec(memory_space=pl.ANY),
            pl.BlockSpec((3, 3 * C, CM), lambda n: (0, 0, 0)),
            pl.BlockSpec((3, 3 * CM, C), lambda n: (0, 0, 0)),
            pl.BlockSpec((1, CM), lambda n: (0, 0)),
            pl.BlockSpec((1, C), lambda n: (0, 0)),
        ],
        out_specs=pl.BlockSpec((1, PR, C), lambda n: (n, 0, 0)),
        out_shape=jax.ShapeDtypeStruct((n_img, PR, C), jnp.float32),
        scratch_shapes=[
            pltpu.VMEM((PR, 3 * C), jnp.bfloat16),
            pltpu.VMEM((PR, CM), jnp.float32),
            pltpu.VMEM((PR, 3 * CM), jnp.bfloat16),
            pltpu.VMEM((R, R), jnp.bfloat16),
            pltpu.SemaphoreType.DMA,
        ],
    )(proposals, pt, x0, w1c, w2c, b1p, b2p)

    y = out.reshape(n_img, 7, 7, R, C).transpose(0, 3, 4, 1, 2)
    return y.reshape(n_img * R, C, 7, 7)
